# baseline (speedup 1.0000x reference)
import functools

import jax
import jax.numpy as jnp
from jax import lax
from jax.experimental import pallas as pl
from jax.experimental.pallas import tpu as pltpu
from jax.experimental.pallas import tpu_sc as plsc

_M = 256
_LOG2M = 8
_L = 16


def _route_scatter_body(NC, NS, N, D, NP,
                        idx_hbm, x_hbm, pos_hbm, te_hbm, xs_hbm,
                        idx_v, hist, nextoff, pos2d, te_v, rows_v, sem):
    NW = NC * NS
    C = N // NW
    NVC = C // _L
    NVT = N // _L
    wid = lax.axis_index("s") * NC + lax.axis_index("c")
    base_tok = wid * C
    v0 = wid * NVC

    iota = lax.iota(jnp.int32, _L)
    ones = jnp.ones((_L,), jnp.int32)
    zeros = jnp.zeros((_L,), jnp.int32)

    pltpu.sync_copy(idx_hbm, idx_v)

    hist[...] = zeros

    def hist_step(i, _):
        v = idx_v[pl.ds(i * _L, _L)]
        plsc.addupdate_scatter(hist, [v], ones)
        return 0

    lax.fori_loop(0, v0, hist_step, 0)
    pref = hist[...]
    lax.fori_loop(v0, NVT, hist_step, 0)
    tot = hist[...]

    padded = ((tot + (_M - 1)) >> _LOG2M) << _LOG2M
    seg_base = plsc.cumsum(padded) - padded
    nextoff[...] = seg_base + pref

    for k in range(NVC):
        v = idx_v[pl.ds((v0 + k) * _L, _L)]
        off = plsc.load_gather(nextoff, [v])
        rank = zeros
        for e in range(8):
            m = v == e
            inc = plsc.cumsum(m.astype(jnp.int32))
            rank = jnp.where(m, inc - 1, rank)
        pos = off + rank
        plsc.addupdate_scatter(nextoff, [v], ones)
        pos2d[k // 2, pl.ds((k % 2) * _L, _L)] = pos

    pltpu.sync_copy(pos2d, pos_hbm.at[pl.ds(wid * (C // 32), C // 32)])

    @pl.when(wid == 0)
    def _te():
        seg_end = seg_base + padded
        TE = te_hbm.shape[0]
        for g in range(TE // _L):
            tv = (iota + g * _L) << _LOG2M
            cnt = jnp.zeros((_L,), jnp.int32)
            for e in range(8):
                end_e = jnp.sum(jnp.where(iota == e, seg_end, 0))
                cnt = cnt + (tv >= end_e).astype(jnp.int32)
            te_v[pl.ds(g * _L, _L)] = jnp.minimum(cnt, 7)
        pltpu.sync_copy(te_v, te_hbm)

    for j in range(C // 32):
        pltpu.sync_copy(x_hbm.at[pl.ds(base_tok + 32 * j, 32)], rows_v)
        pltpu.async_copy(rows_v, xs_hbm.at[pos2d.at[j]], sem).wait()


def _unpermute_body(NC, NS, N, O,
                    pos_hbm, ys_hbm, y_hbm,
                    posk, rows_v, sem):
    NW = NC * NS
    C = N // NW
    wid = lax.axis_index("s") * NC + lax.axis_index("c")
    base_tok = wid * C
    JB = C // 64
    pltpu.sync_copy(pos_hbm.at[pl.ds(wid * JB, JB)], posk)
    for j in range(JB):
        pltpu.async_copy(ys_hbm.at[posk.at[j]], rows_v, sem).wait()
        pltpu.sync_copy(rows_v, y_hbm.at[pl.ds(base_tok + 64 * j, 64)])


def _mm_body(te_ref, xs_ref, w_ref, b_ref, o_ref):
    xb = xs_ref[...].astype(jnp.bfloat16)
    o_ref[...] = jax.lax.dot_general(
        xb, w_ref[0], (((1,), (1,)), ((), ())),
        preferred_element_type=jnp.float32) + b_ref[0]


def kernel(x, index, W, b):
    N, D = x.shape
    E, O, _ = W.shape
    NP = N + E * _M
    T = NP // _M
    TE = ((T + _L - 1) // _L) * _L

    mesh = plsc.VectorSubcoreMesh(core_axis_name="c", subcore_axis_name="s")
    NC, NS = mesh.num_cores, mesh.num_subcores
    NW = NC * NS
    C = N // NW

    route_scatter = pl.kernel(
        functools.partial(_route_scatter_body, NC, NS, N, D, NP),
        out_type=(
            jax.ShapeDtypeStruct((N // 32, 32), jnp.int32),
            jax.ShapeDtypeStruct((TE,), jnp.int32),
            jax.ShapeDtypeStruct((NP, D), jnp.float32),
        ),
        mesh=mesh,
        compiler_params=pltpu.CompilerParams(needs_layout_passes=False),
        scratch_types=[
            pltpu.VMEM((N,), jnp.int32),
            pltpu.VMEM((_L,), jnp.int32),
            pltpu.VMEM((_L,), jnp.int32),
            pltpu.VMEM((C // 32, 32), jnp.int32),
            pltpu.VMEM((TE,), jnp.int32),
            pltpu.VMEM((32, D), jnp.float32),
            pltpu.SemaphoreType.DMA,
        ],
    )

    unpermute = pl.kernel(
        functools.partial(_unpermute_body, NC, NS, N, O),
        out_type=jax.ShapeDtypeStruct((N, O), jnp.float32),
        mesh=mesh,
        scratch_types=[
            pltpu.VMEM((C // 64, 64), jnp.int32),
            pltpu.VMEM((64, O), jnp.float32),
            pltpu.SemaphoreType.DMA,
        ],
    )

    idx32 = index.astype(jnp.int32)
    pos, te, xs = route_scatter(idx32, x)

    ys = pl.pallas_call(
        _mm_body,
        grid_spec=pltpu.PrefetchScalarGridSpec(
            num_scalar_prefetch=1,
            grid=(T,),
            in_specs=[
                pl.BlockSpec((_M, D), lambda t, te_ref: (t, 0)),
                pl.BlockSpec((1, O, D), lambda t, te_ref: (te_ref[t], 0, 0)),
                pl.BlockSpec((1, 1, O), lambda t, te_ref: (te_ref[t], 0, 0)),
            ],
            out_specs=pl.BlockSpec((_M, O), lambda t, te_ref: (t, 0)),
        ),
        out_shape=jax.ShapeDtypeStruct((NP, O), jnp.float32),
    )(te, xs, W.astype(jnp.bfloat16), b.reshape(E, 1, O))

    y = unpermute(pos.reshape(N // 64, 64), ys)
    return y


# speedup vs baseline: 1.0612x; 1.0612x over previous
import functools

import jax
import jax.numpy as jnp
from jax import lax
from jax.experimental import pallas as pl
from jax.experimental.pallas import tpu as pltpu
from jax.experimental.pallas import tpu_sc as plsc

_M = 256
_LOG2M = 8
_L = 16


def _route_scatter_body(NC, NS, N, D, NP,
                        idx_hbm, x_hbm, pos_hbm, te_hbm, xs_hbm,
                        idx_v, hist, nextoff, pos2d, te_v, rows0, rows1,
                        sem_ld, sem_sc):
    NW = NC * NS
    C = N // NW
    NVC = C // _L
    NVT = N // _L
    wid = lax.axis_index("s") * NC + lax.axis_index("c")
    base_tok = wid * C
    v0 = wid * NVC

    iota = lax.iota(jnp.int32, _L)
    ones = jnp.ones((_L,), jnp.int32)
    zeros = jnp.zeros((_L,), jnp.int32)

    pltpu.sync_copy(idx_hbm, idx_v)

    hist[...] = zeros

    def hist_step(i, _):
        v = idx_v[pl.ds(i * _L, _L)]
        plsc.addupdate_scatter(hist, [v], ones)
        return 0

    lax.fori_loop(0, v0, hist_step, 0)
    pref = hist[...]
    lax.fori_loop(v0, NVT, hist_step, 0)
    tot = hist[...]

    padded = ((tot + (_M - 1)) >> _LOG2M) << _LOG2M
    seg_base = plsc.cumsum(padded) - padded
    nextoff[...] = seg_base + pref

    for k in range(NVC):
        v = idx_v[pl.ds((v0 + k) * _L, _L)]
        off = plsc.load_gather(nextoff, [v])
        rank = zeros
        for e in range(8):
            m = v == e
            inc = plsc.cumsum(m.astype(jnp.int32))
            rank = jnp.where(m, inc - 1, rank)
        plsc.addupdate_scatter(nextoff, [v], ones)
        pos2d[k] = off + rank

    pltpu.sync_copy(pos2d, pos_hbm.at[pl.ds(wid * NVC, NVC)])

    @pl.when(wid == 0)
    def _te():
        seg_end = seg_base + padded
        TE = te_hbm.shape[0]
        for g in range(TE // _L):
            tv = (iota + g * _L) << _LOG2M
            cnt = jnp.zeros((_L,), jnp.int32)
            for e in range(8):
                end_e = jnp.sum(jnp.where(iota == e, seg_end, 0))
                cnt = cnt + (tv >= end_e).astype(jnp.int32)
            te_v[pl.ds(g * _L, _L)] = jnp.minimum(cnt, 7)
        pltpu.sync_copy(te_v, te_hbm)

    bufs = (rows0, rows1)
    loads = [None] * NVC
    scats = [None] * NVC
    loads[0] = pltpu.async_copy(
        x_hbm.at[pl.ds(base_tok, _L)], bufs[0], sem_ld.at[0])
    for j in range(NVC):
        if j + 1 < NVC:
            if j >= 1:
                scats[j - 1].wait()
            loads[j + 1] = pltpu.async_copy(
                x_hbm.at[pl.ds(base_tok + _L * (j + 1), _L)],
                bufs[(j + 1) % 2], sem_ld.at[(j + 1) % 2])
        loads[j].wait()
        scats[j] = pltpu.async_copy(
            bufs[j % 2], xs_hbm.at[pos2d.at[j]], sem_sc.at[j % 2])
    scats[NVC - 2].wait()
    scats[NVC - 1].wait()


def _unpermute_body(NC, NS, N, O,
                    pos_hbm, ys_hbm, y_hbm,
                    posk, rows0, rows1, sem_ld, sem_sc):
    NW = NC * NS
    C = N // NW
    wid = lax.axis_index("s") * NC + lax.axis_index("c")
    base_tok = wid * C
    JB = C // 32
    bufs = (rows0, rows1)
    pltpu.sync_copy(pos_hbm.at[pl.ds(wid * JB, JB)], posk)
    gets = [None] * JB
    puts = [None] * JB
    gets[0] = pltpu.async_copy(ys_hbm.at[posk.at[0]], bufs[0], sem_ld.at[0])
    for j in range(JB):
        if j + 1 < JB:
            if j >= 1:
                puts[j - 1].wait()
            gets[j + 1] = pltpu.async_copy(
                ys_hbm.at[posk.at[j + 1]], bufs[(j + 1) % 2],
                sem_ld.at[(j + 1) % 2])
        gets[j].wait()
        puts[j] = pltpu.async_copy(
            bufs[j % 2], y_hbm.at[pl.ds(base_tok + 32 * j, 32)],
            sem_sc.at[j % 2])
    puts[JB - 2].wait()
    puts[JB - 1].wait()


def _mm_body(te_ref, xs_ref, w_ref, b_ref, o_ref, wb_ref, last_ref):
    t = pl.program_id(0)
    te = te_ref[t]

    @pl.when((t == 0) | (te != last_ref[0]))
    def _cast_w():
        wb_ref[...] = w_ref[0].astype(jnp.bfloat16)
        last_ref[0] = te

    xb = xs_ref[...].astype(jnp.bfloat16)
    acc = jax.lax.dot_general(
        xb, wb_ref[...], (((1,), (1,)), ((), ())),
        preferred_element_type=jnp.float32)
    o_ref[...] = acc + b_ref[0]


def kernel(x, index, W, b):
    N, D = x.shape
    E, O, _ = W.shape
    NP = N + E * _M
    T = NP // _M
    TE = ((T + _L - 1) // _L) * _L

    mesh = plsc.VectorSubcoreMesh(core_axis_name="c", subcore_axis_name="s")
    NC, NS = mesh.num_cores, mesh.num_subcores
    NW = NC * NS
    C = N // NW

    route_scatter = pl.kernel(
        functools.partial(_route_scatter_body, NC, NS, N, D, NP),
        out_type=(
            jax.ShapeDtypeStruct((N // _L, _L), jnp.int32),
            jax.ShapeDtypeStruct((TE,), jnp.int32),
            jax.ShapeDtypeStruct((NP, D), jnp.float32),
        ),
        mesh=mesh,
        compiler_params=pltpu.CompilerParams(needs_layout_passes=False),
        scratch_types=[
            pltpu.VMEM((N,), jnp.int32),
            pltpu.VMEM((_L,), jnp.int32),
            pltpu.VMEM((_L,), jnp.int32),
            pltpu.VMEM((C // _L, _L), jnp.int32),
            pltpu.VMEM((TE,), jnp.int32),
            pltpu.VMEM((_L, D), jnp.float32),
            pltpu.VMEM((_L, D), jnp.float32),
            pltpu.SemaphoreType.DMA((2,)),
            pltpu.SemaphoreType.DMA((2,)),
        ],
    )

    unpermute = pl.kernel(
        functools.partial(_unpermute_body, NC, NS, N, O),
        out_type=jax.ShapeDtypeStruct((N, O), jnp.float32),
        mesh=mesh,
        compiler_params=pltpu.CompilerParams(needs_layout_passes=False),
        scratch_types=[
            pltpu.VMEM((C // 32, 32), jnp.int32),
            pltpu.VMEM((32, O), jnp.float32),
            pltpu.VMEM((32, O), jnp.float32),
            pltpu.SemaphoreType.DMA((2,)),
            pltpu.SemaphoreType.DMA((2,)),
        ],
    )

    idx32 = index.astype(jnp.int32)
    pos, te, xs = route_scatter(idx32, x)

    ys = pl.pallas_call(
        _mm_body,
        grid_spec=pltpu.PrefetchScalarGridSpec(
            num_scalar_prefetch=1,
            grid=(T,),
            in_specs=[
                pl.BlockSpec((_M, D), lambda t, te_ref: (t, 0)),
                pl.BlockSpec((1, O, D), lambda t, te_ref: (te_ref[t], 0, 0)),
                pl.BlockSpec((1, 1, O), lambda t, te_ref: (te_ref[t], 0, 0)),
            ],
            out_specs=pl.BlockSpec((_M, O), lambda t, te_ref: (t, 0)),
            scratch_shapes=[
                pltpu.VMEM((O, D), jnp.bfloat16),
                pltpu.SMEM((1,), jnp.int32),
            ],
        ),
        out_shape=jax.ShapeDtypeStruct((NP, O), jnp.float32),
    )(te, xs, W, b.reshape(E, 1, O))

    y = unpermute(pos.reshape(N // 32, 32), ys)
    return y
